# baseline (speedup 1.0000x reference)
import jax
import jax.numpy as jnp
from jax import lax
from jax.experimental import pallas as pl
from jax.experimental.pallas import tpu as pltpu
from jax.experimental.pallas import tpu_sc as plsc

_ROW = 128
_NBUF = 4
_CHUNK = 160


def _gather_rows(table_p, idx_flat):
    num_idx = idx_flat.shape[0]
    info = plsc.get_sparse_core_info()
    nw = info.num_cores * info.num_subcores
    per_w = num_idx // nw
    nchunks = per_w // _CHUNK
    assert per_w % _CHUNK == 0 and nchunks % _NBUF == 0 and nchunks >= 4
    mesh = plsc.VectorSubcoreMesh(
        core_axis_name="core", subcore_axis_name="subcore"
    )

    @pl.kernel(
        out_type=jax.ShapeDtypeStruct((num_idx, _ROW), table_p.dtype),
        mesh=mesh,
        compiler_params=pltpu.CompilerParams(use_tc_tiling_on_sc=True),
        scratch_types=[
            pltpu.VMEM((per_w,), jnp.int32),
            pltpu.VMEM((_NBUF, _CHUNK, _ROW), jnp.float32),
        ]
        + [pltpu.SemaphoreType.DMA] * (2 * _NBUF),
    )
    def k(table_hbm, idx_hbm, out_hbm, idx_v, rows_v, *sems):
        gsem, osem = sems[:_NBUF], sems[_NBUF:]
        wid = lax.axis_index("subcore") * info.num_cores + lax.axis_index(
            "core"
        )
        base = wid * per_w
        pltpu.sync_copy(idx_hbm.at[pl.ds(base, per_w)], idx_v)

        def g_copy(i, slot):
            return pltpu.make_async_copy(
                table_hbm.at[idx_v.at[pl.ds(i * _CHUNK, _CHUNK)]],
                rows_v.at[slot],
                gsem[slot],
            )

        def o_copy(i, slot):
            return pltpu.make_async_copy(
                rows_v.at[slot],
                out_hbm.at[pl.ds(base + i * _CHUNK, _CHUNK)],
                osem[slot],
            )

        g_copy(0, 0).start()
        g_copy(1, 1).start()

        @pl.loop(0, nchunks // _NBUF)
        def _(g):
            for b in range(_NBUF):
                i = g * _NBUF + b
                s2 = (b + 2) % _NBUF
                g_copy(i, b).wait()
                o_copy(i, b).start()

                @pl.when(i >= 2)
                def _():
                    o_copy(i - 2, s2).wait()

                @pl.when(i + 2 < nchunks)
                def _():
                    g_copy(i + 2, s2).start()

        o_copy(nchunks - 2, (nchunks - 2) % _NBUF).wait()
        o_copy(nchunks - 1, (nchunks - 1) % _NBUF).wait()

    return k(table_p, idx_flat)


def kernel(x, table):
    batch, hist = x.shape
    dim = table.shape[1]
    idx_flat = x.astype(jnp.int32).reshape(-1)
    table_p = jnp.pad(table, ((0, 0), (0, _ROW - dim)))
    rows = _gather_rows(table_p, idx_flat)
    return rows[:, :dim].reshape(batch, hist, dim)


# speedup vs baseline: 1.3095x; 1.3095x over previous
import jax
import jax.numpy as jnp
from jax import lax
from jax.experimental import pallas as pl
from jax.experimental.pallas import tpu as pltpu
from jax.experimental.pallas import tpu_sc as plsc

_ROW = 128
_HISTPAD = 56
_NBUF = 4
_CHUNK = 112


def _gather_rows(table_p, idx_flat):
    num_idx = idx_flat.shape[0]
    info = plsc.get_sparse_core_info()
    nw = info.num_cores * info.num_subcores
    per_w = num_idx // nw
    nchunks = per_w // _CHUNK
    assert per_w % _CHUNK == 0 and nchunks % _NBUF == 0 and nchunks >= 4
    mesh = plsc.VectorSubcoreMesh(
        core_axis_name="core", subcore_axis_name="subcore"
    )

    @pl.kernel(
        out_type=jax.ShapeDtypeStruct((num_idx, _ROW), table_p.dtype),
        mesh=mesh,
        compiler_params=pltpu.CompilerParams(use_tc_tiling_on_sc=True),
        scratch_types=[
            pltpu.VMEM((per_w,), jnp.int32),
            pltpu.VMEM((_NBUF, _CHUNK, _ROW), jnp.float32),
        ]
        + [pltpu.SemaphoreType.DMA] * (2 * _NBUF),
    )
    def k(table_hbm, idx_hbm, out_hbm, idx_v, rows_v, *sems):
        gsem, osem = sems[:_NBUF], sems[_NBUF:]
        wid = lax.axis_index("subcore") * info.num_cores + lax.axis_index(
            "core"
        )
        base = wid * per_w
        pltpu.sync_copy(idx_hbm.at[pl.ds(base, per_w)], idx_v)

        def g_copy(i, slot):
            return pltpu.make_async_copy(
                table_hbm.at[idx_v.at[pl.ds(i * _CHUNK, _CHUNK)]],
                rows_v.at[slot],
                gsem[slot],
            )

        def o_copy(i, slot):
            return pltpu.make_async_copy(
                rows_v.at[slot],
                out_hbm.at[pl.ds(base + i * _CHUNK, _CHUNK)],
                osem[slot],
            )

        g_copy(0, 0).start()
        g_copy(1, 1).start()

        @pl.loop(0, nchunks // _NBUF)
        def _(g):
            for b in range(_NBUF):
                i = g * _NBUF + b
                s2 = (b + 2) % _NBUF
                g_copy(i, b).wait()
                o_copy(i, b).start()

                @pl.when(i >= 2)
                def _():
                    o_copy(i - 2, s2).wait()

                @pl.when(i + 2 < nchunks)
                def _():
                    g_copy(i + 2, s2).start()

        o_copy(nchunks - 2, (nchunks - 2) % _NBUF).wait()
        o_copy(nchunks - 1, (nchunks - 1) % _NBUF).wait()

    return k(table_p, idx_flat)


def kernel(x, table):
    batch, hist = x.shape
    dim = table.shape[1]
    x_pad = jnp.pad(
        x.astype(jnp.int32), ((0, 0), (0, _HISTPAD - hist)), mode="edge"
    )
    idx_flat = x_pad.reshape(-1)
    table_p = jnp.pad(table, ((0, 0), (0, _ROW - dim)))
    rows = _gather_rows(table_p, idx_flat)
    return rows.reshape(batch, _HISTPAD, _ROW)[:, :hist, :dim]
